# Initial kernel scaffold; baseline (speedup 1.0000x reference)
import jax
import jax.numpy as jnp
from jax.experimental import pallas as pl


def kernel(x_num, x_cat, W_num, b_num, tables, gamma, beta):
    raise NotImplementedError("write your pallas kernel here")


# speedup vs baseline: 3.8985x; 3.8985x over previous
import functools

import jax
import jax.numpy as jnp
from jax import lax
from jax.experimental import pallas as pl
from jax.experimental.pallas import tpu as pltpu
from jax.experimental.pallas import tpu_sc as plsc

B, T, Cn, Cc, V, D = 1024, 50, 16, 26, 100000, 32
N = B * T
NC, NS = 2, 16
NW = NC * NS
TOK_W = N // NW
CH = 64
IDX_CH = CH * Cc
STREAM = 128
NSTREAM = IDX_CH // STREAM
NCHUNK = TOK_W // CH
PAT = 208


def _sc_embed_sum(tables_flat, xcat_flat, off_pat):
    mesh = plsc.VectorSubcoreMesh(core_axis_name="c", subcore_axis_name="s")

    @functools.partial(
        pl.kernel,
        mesh=mesh,
        out_type=jax.ShapeDtypeStruct((N, D), jnp.float32),
        compiler_params=pltpu.CompilerParams(use_tc_tiling_on_sc=False),
        scratch_types=[
            pltpu.VMEM((IDX_CH,), jnp.int32),
            pltpu.VMEM((IDX_CH, D), jnp.float32),
            pltpu.VMEM((CH, D), jnp.float32),
            pltpu.VMEM((PAT,), jnp.int32),
            pltpu.SemaphoreType.DMA,
        ],
    )
    def k(tab_hbm, idx_hbm, off_hbm, out_hbm, idx_v, rows_v, acc_v, off_v, sem):
        wid = lax.axis_index("s") * NC + lax.axis_index("c")
        pltpu.sync_copy(off_hbm, off_v)

        def chunk_body(ci, carry):
            tok0 = wid * TOK_W + ci * CH
            base = tok0 * Cc
            pltpu.sync_copy(idx_hbm.at[pl.ds(base, IDX_CH)], idx_v)

            def add_body(rep, c2):
                for kk in range(PAT // 16):
                    sl = pl.ds(rep * PAT + kk * 16, 16)
                    idx_v[sl] = idx_v[sl] + off_v[pl.ds(kk * 16, 16)]
                return c2

            lax.fori_loop(0, IDX_CH // PAT, add_body, 0)

            copies = [
                pltpu.async_copy(
                    tab_hbm.at[idx_v.at[pl.ds(s * STREAM, STREAM)]],
                    rows_v.at[pl.ds(s * STREAM, STREAM)],
                    sem,
                )
                for s in range(NSTREAM)
            ]
            for cp in copies:
                cp.wait()

            def tok_body(n, c2):
                r0 = n * Cc
                h0 = rows_v[r0, pl.ds(0, 16)]
                h1 = rows_v[r0, pl.ds(16, 16)]
                for c in range(1, Cc):
                    h0 = h0 + rows_v[r0 + c, pl.ds(0, 16)]
                    h1 = h1 + rows_v[r0 + c, pl.ds(16, 16)]
                acc_v[n, pl.ds(0, 16)] = h0
                acc_v[n, pl.ds(16, 16)] = h1
                return c2

            lax.fori_loop(0, CH, tok_body, 0)
            pltpu.sync_copy(acc_v, out_hbm.at[pl.ds(tok0, CH)])
            return carry

        lax.fori_loop(0, NCHUNK, chunk_body, 0)

    return k(tables_flat, xcat_flat, off_pat)


def _tc_finish(x_num2, W_num, b_num2, cat_sum, gamma2, beta2):
    BLK = 2048

    def body(x_ref, w_ref, b_ref, s_ref, g_ref, bt_ref, o_ref):
        num = jnp.dot(x_ref[...], w_ref[...], preferred_element_type=jnp.float32)
        x = num + b_ref[...] + s_ref[...] * (1.0 / Cc)
        m = jnp.mean(x, axis=-1, keepdims=True)
        v = jnp.mean((x - m) ** 2, axis=-1, keepdims=True)
        o_ref[...] = (x - m) * lax.rsqrt(v + 1e-5) * g_ref[...] + bt_ref[...]

    return pl.pallas_call(
        body,
        grid=(N // BLK,),
        in_specs=[
            pl.BlockSpec((BLK, Cn), lambda i: (i, 0)),
            pl.BlockSpec((Cn, D), lambda i: (0, 0)),
            pl.BlockSpec((1, D), lambda i: (0, 0)),
            pl.BlockSpec((BLK, D), lambda i: (i, 0)),
            pl.BlockSpec((1, D), lambda i: (0, 0)),
            pl.BlockSpec((1, D), lambda i: (0, 0)),
        ],
        out_specs=pl.BlockSpec((BLK, D), lambda i: (i, 0)),
        out_shape=jax.ShapeDtypeStruct((N, D), jnp.float32),
    )(x_num2, W_num, b_num2, cat_sum, gamma2, beta2)


def kernel(x_num, x_cat, W_num, b_num, tables, gamma, beta):
    xc = x_cat.astype(jnp.int32).reshape(N * Cc)
    tabf = tables.reshape(Cc * V, D)
    off = jnp.tile(jnp.arange(Cc, dtype=jnp.int32) * V, PAT // Cc)
    cat_sum = _sc_embed_sum(tabf, xc, off)
    out = _tc_finish(
        x_num.reshape(N, Cn),
        W_num,
        b_num.reshape(1, D),
        cat_sum,
        gamma.reshape(1, D),
        beta.reshape(1, D),
    )
    return out.reshape(B, T, D)


# speedup vs baseline: 4.1245x; 1.0580x over previous
import functools

import jax
import jax.numpy as jnp
from jax import lax
from jax.experimental import pallas as pl
from jax.experimental.pallas import tpu as pltpu
from jax.experimental.pallas import tpu_sc as plsc

B, T, Cn, Cc, V, D = 1024, 50, 16, 26, 100000, 32
N = B * T
NC, NS = 2, 16
NW = NC * NS
TOK_W = N // NW
CH = 80
NCHUNK = TOK_W // CH
NSTREAM = Cc * NCHUNK


def _sc_embed_sum(tables_flat, xcat_t):
    mesh = plsc.VectorSubcoreMesh(core_axis_name="c", subcore_axis_name="s")

    @functools.partial(
        pl.kernel,
        mesh=mesh,
        out_type=jax.ShapeDtypeStruct((N, D), jnp.float32),
        compiler_params=pltpu.CompilerParams(use_tc_tiling_on_sc=False),
        scratch_types=[
            pltpu.VMEM((Cc, TOK_W), jnp.int32),
            pltpu.VMEM((TOK_W, D), jnp.float32),
            pltpu.SemaphoreType.DMA,
        ],
    )
    def k(tab_hbm, idx_hbm, out_hbm, idx_v, acc_v, sem):
        wid = lax.axis_index("s") * NC + lax.axis_index("c")
        base = wid * TOK_W
        pltpu.sync_copy(idx_hbm.at[:, pl.ds(base, TOK_W)], idx_v)

        zero = jnp.zeros((16,), jnp.float32)

        def zero_body(n, c2):
            acc_v[n, pl.ds(0, 16)] = zero
            acc_v[n, pl.ds(16, 16)] = zero
            return c2

        lax.fori_loop(0, TOK_W, zero_body, 0)

        def off_body(c, c2):
            off = c * V

            def off_inner(j, c3):
                for kk in range(10):
                    sl = pl.ds(j * 160 + kk * 16, 16)
                    idx_v[c, sl] = idx_v[c, sl] + off
                return c3

            lax.fori_loop(0, TOK_W // 160, off_inner, 0)
            return c2

        lax.fori_loop(0, Cc, off_body, 0)

        def fire_body(s, c2):
            c = s // NCHUNK
            ci = s % NCHUNK
            pltpu.async_copy(
                tab_hbm.at[idx_v.at[c, pl.ds(ci * CH, CH)]],
                acc_v.at[pl.ds(ci * CH, CH)],
                sem,
                add=True,
            )
            return c2

        lax.fori_loop(0, NSTREAM, fire_body, 0)

        def drain_body(s, c2):
            pltpu.make_async_copy(
                tab_hbm.at[idx_v.at[0, pl.ds(0, CH)]],
                acc_v.at[pl.ds(0, CH)],
                sem,
            ).wait()
            return c2

        lax.fori_loop(0, NSTREAM, drain_body, 0)

        pltpu.sync_copy(acc_v, out_hbm.at[pl.ds(base, TOK_W)])

    return k(tables_flat, xcat_t)


def _tc_finish(x_num2, W_num, b_num2, cat_sum, gamma2, beta2):
    BLK = 2048

    def body(x_ref, w_ref, b_ref, s_ref, g_ref, bt_ref, o_ref):
        num = jnp.dot(x_ref[...], w_ref[...], preferred_element_type=jnp.float32)
        x = num + b_ref[...] + s_ref[...] * (1.0 / Cc)
        m = jnp.mean(x, axis=-1, keepdims=True)
        v = jnp.mean((x - m) ** 2, axis=-1, keepdims=True)
        o_ref[...] = (x - m) * lax.rsqrt(v + 1e-5) * g_ref[...] + bt_ref[...]

    return pl.pallas_call(
        body,
        grid=(N // BLK,),
        in_specs=[
            pl.BlockSpec((BLK, Cn), lambda i: (i, 0)),
            pl.BlockSpec((Cn, D), lambda i: (0, 0)),
            pl.BlockSpec((1, D), lambda i: (0, 0)),
            pl.BlockSpec((BLK, D), lambda i: (i, 0)),
            pl.BlockSpec((1, D), lambda i: (0, 0)),
            pl.BlockSpec((1, D), lambda i: (0, 0)),
        ],
        out_specs=pl.BlockSpec((BLK, D), lambda i: (i, 0)),
        out_shape=jax.ShapeDtypeStruct((N, D), jnp.float32),
    )(x_num2, W_num, b_num2, cat_sum, gamma2, beta2)


def kernel(x_num, x_cat, W_num, b_num, tables, gamma, beta):
    xcat_t = x_cat.astype(jnp.int32).reshape(N, Cc).T
    tabf = tables.reshape(Cc * V, D)
    cat_sum = _sc_embed_sum(tabf, xcat_t)
    out = _tc_finish(
        x_num.reshape(N, Cn),
        W_num,
        b_num.reshape(1, D),
        cat_sum,
        gamma.reshape(1, D),
        beta.reshape(1, D),
    )
    return out.reshape(B, T, D)
